# Initial kernel scaffold; baseline (speedup 1.0000x reference)
import jax
import jax.numpy as jnp
from jax.experimental import pallas as pl


def kernel(x, z, edge_index, z1edge_index, z2edge_index, z3edge_index, edge_attr, pickable, W_enc, b_enc, W_c1, b_c1, W_c2, b_c2, W_x1, b_x1, W_lin, b_lin):
    raise NotImplementedError("write your pallas kernel here")


# speedup vs baseline: 22.2380x; 22.2380x over previous
import functools

import jax
import jax.numpy as jnp
from jax import lax
from jax.experimental import pallas as pl
from jax.experimental.pallas import tpu as pltpu
from jax.experimental.pallas import tpu_sc as plsc

D = 16
CH = 128
NBUF = 4
GRP = CH * NBUF
NC = 2
NS = 16
NW = NC * NS


def _round_up(a: int, b: int) -> int:
    return (a + b - 1) // b * b


def _make_agg(n_nodes: int, g_per_w: int, nacc: int):
    mesh = plsc.VectorSubcoreMesh(core_axis_name="c", subcore_axis_name="s")
    zrows = nacc // NS

    def body(h_hbm, srcr_hbm, dstr_hbm, zeros_hbm, out_hbm,
             sidx, didx, rows, acc, sem):
        c = lax.axis_index("c")
        s = lax.axis_index("s")
        w = c * NS + s

        pltpu.sync_copy(zeros_hbm.at[pl.ds(s * zrows, zrows)],
                        acc.at[pl.ds(s * zrows, zrows)])
        plsc.subcore_barrier()

        def grp_body(g, carry):
            chunk0 = (w * g_per_w + g) * NBUF
            pltpu.sync_copy(srcr_hbm.at[pl.ds(chunk0, NBUF)], sidx)
            pltpu.sync_copy(dstr_hbm.at[pl.ds(chunk0, NBUF)], didx)
            cps = [pltpu.async_copy(h_hbm.at[sidx.at[b]], rows.at[b], sem)
                   for b in range(NBUF)]
            for cp in cps:
                cp.wait()
            for b in range(NBUF):
                pltpu.sync_copy(rows.at[b], acc.at[didx.at[b]], add=True)
            return carry

        lax.fori_loop(0, g_per_w, grp_body, 0)
        plsc.subcore_barrier()

        pltpu.sync_copy(acc.at[pl.ds(s * zrows, zrows)],
                        out_hbm.at[c].at[pl.ds(s * zrows, zrows)])

    return pl.kernel(
        body,
        out_type=jax.ShapeDtypeStruct((NC, nacc, D), jnp.float32),
        mesh=mesh,
        scratch_types=[
            pltpu.VMEM((NBUF, CH), jnp.int32),
            pltpu.VMEM((NBUF, CH), jnp.int32),
            pltpu.VMEM((NBUF, CH, D), jnp.float32),
            pltpu.VMEM_SHARED((nacc, D), jnp.float32),
            pltpu.SemaphoreType.DMA,
        ],
        compiler_params=pltpu.CompilerParams(use_tc_tiling_on_sc=False),
    )


def _make_pick(n_out_chunks_per_w: int):
    mesh = plsc.VectorSubcoreMesh(core_axis_name="c", subcore_axis_name="s")
    g_per_w = n_out_chunks_per_w // NBUF

    def body(h_hbm, idxr_hbm, out_hbm, sidx, rows, sem):
        c = lax.axis_index("c")
        s = lax.axis_index("s")
        w = c * NS + s

        def grp_body(g, carry):
            chunk0 = (w * g_per_w + g) * NBUF
            pltpu.sync_copy(idxr_hbm.at[pl.ds(chunk0, NBUF)], sidx)
            cps = [pltpu.async_copy(h_hbm.at[sidx.at[b]],
                                    rows.at[pl.ds(b * CH, CH)], sem)
                   for b in range(NBUF)]
            for cp in cps:
                cp.wait()
            pltpu.sync_copy(rows, out_hbm.at[pl.ds(chunk0 * CH, GRP)])
            return carry

        lax.fori_loop(0, g_per_w, grp_body, 0)

    def make(pp):
        return pl.kernel(
            body,
            out_type=jax.ShapeDtypeStruct((pp, D), jnp.float32),
            mesh=mesh,
            scratch_types=[
                pltpu.VMEM((NBUF, CH), jnp.int32),
                pltpu.VMEM((GRP, D), jnp.float32),
                pltpu.SemaphoreType.DMA,
            ],
            compiler_params=pltpu.CompilerParams(use_tc_tiling_on_sc=False),
        )

    return make


def _enc_body(z_ref, w_ref, c_ref, o_ref):
    o_ref[...] = (jnp.dot(z_ref[...], w_ref[...],
                          preferred_element_type=jnp.float32) + c_ref[...])


def _lin_body(p_ref, w_ref, c_ref, o_ref, *, mean, relu_on):
    a = p_ref[0] + p_ref[1]
    if mean:
        a = a / jnp.maximum(a[:, 10:11], 1.0)
    y = jnp.dot(a, w_ref[...], preferred_element_type=jnp.float32) + c_ref[...]
    if relu_on:
        y = jnp.maximum(y, 0.0)
    o_ref[...] = y


def _dec_body(p_ref, w_ref, c_ref, o_ref):
    y = (jnp.dot(p_ref[...], w_ref[...],
                 preferred_element_type=jnp.float32) + c_ref[...])
    m = jnp.max(y, axis=1, keepdims=True)
    e = jnp.exp(y - m)
    o_ref[...] = (e / jnp.sum(e, axis=1, keepdims=True))[:, :10]


def _pad_w(w):
    wp = jnp.zeros((D, D), jnp.float32)
    return wp.at[:w.shape[0], :w.shape[1]].set(w)


def _cvec(b, ones_lane=True):
    c = jnp.zeros((1, D), jnp.float32).at[0, :b.shape[0]].set(b)
    if ones_lane:
        c = c.at[0, 10].set(1.0)
    return c


def kernel(x, z, edge_index, z1edge_index, z2edge_index, z3edge_index,
           edge_attr, pickable, W_enc, b_enc, W_c1, b_c1, W_c2, b_c2,
           W_x1, b_x1, W_lin, b_lin):
    n, zf = z.shape
    e = z1edge_index.shape[1]
    p = pickable.shape[0]
    assert n % NS == 0

    g_per_w = _round_up(e, NW * GRP) // (NW * GRP)
    ep = NW * g_per_w * GRP
    src = z1edge_index[0]
    dst = z1edge_index[1]
    if ep != e:
        src = jnp.concatenate([src, jnp.zeros((ep - e,), jnp.int32)])
        dst = jnp.concatenate([dst, jnp.full((ep - e,), n, jnp.int32)])
    srcr = src.reshape(ep // CH, CH)
    dstr = dst.reshape(ep // CH, CH)

    nacc = _round_up(n + 1, NS * 8)
    zeros_acc = jnp.zeros((nacc, D), jnp.float32)

    wenc = jnp.zeros((zf, D), jnp.float32).at[:, :W_enc.shape[1]].set(W_enc)
    cenc = _cvec(b_enc)
    w1, c1 = _pad_w(W_c1), _cvec(b_c1)
    w2, c2 = _pad_w(W_c2), _cvec(b_c2)
    wx, cx = _pad_w(W_x1), _cvec(b_x1)
    wl = _pad_w(W_lin)
    cl = jnp.full((1, D), -1e30, jnp.float32).at[0, :b_lin.shape[0]].set(b_lin)

    br = 2000
    nblk = n // br
    f32 = jnp.float32

    enc = pl.pallas_call(
        _enc_body,
        grid=(nblk,),
        in_specs=[pl.BlockSpec((br, zf), lambda i: (i, 0)),
                  pl.BlockSpec((zf, D), lambda i: (0, 0)),
                  pl.BlockSpec((1, D), lambda i: (0, 0))],
        out_specs=pl.BlockSpec((br, D), lambda i: (i, 0)),
        out_shape=jax.ShapeDtypeStruct((n, D), f32),
    )

    def lin(mean, relu_on):
        return pl.pallas_call(
            functools.partial(_lin_body, mean=mean, relu_on=relu_on),
            grid=(nblk,),
            in_specs=[pl.BlockSpec((NC, br, D), lambda i: (0, i, 0)),
                      pl.BlockSpec((D, D), lambda i: (0, 0)),
                      pl.BlockSpec((1, D), lambda i: (0, 0))],
            out_specs=pl.BlockSpec((br, D), lambda i: (i, 0)),
            out_shape=jax.ShapeDtypeStruct((n, D), f32),
        )

    agg = _make_agg(n, g_per_w, nacc)

    h0 = enc(z, wenc, cenc)
    h1 = lin(False, True)(agg(h0, srcr, dstr, zeros_acc), w1, c1)
    h2 = lin(False, False)(agg(h1, srcr, dstr, zeros_acc), w2, c2)
    h3 = lin(True, True)(agg(h2, srcr, dstr, zeros_acc), wx, cx)

    pk_chunks_per_w = _round_up(p, NW * GRP) // (NW * CH)
    pp = pk_chunks_per_w * NW * CH
    pidx = pickable
    if pp != p:
        pidx = jnp.concatenate([pidx, jnp.zeros((pp - p,), jnp.int32)])
    pidxr = pidx.reshape(pp // CH, CH)
    picked = _make_pick(pk_chunks_per_w)(pp)(h3, pidxr)

    pbr = 2000
    pblk = p // pbr
    out = pl.pallas_call(
        _dec_body,
        grid=(pblk,),
        in_specs=[pl.BlockSpec((pbr, D), lambda i: (i, 0)),
                  pl.BlockSpec((D, D), lambda i: (0, 0)),
                  pl.BlockSpec((1, D), lambda i: (0, 0))],
        out_specs=pl.BlockSpec((pbr, 10), lambda i: (i, 0)),
        out_shape=jax.ShapeDtypeStruct((p, 10), f32),
    )(picked, wl, cl)
    return out


# speedup vs baseline: 34.3245x; 1.5435x over previous
import functools

import jax
import jax.numpy as jnp
from jax import lax
from jax.experimental import pallas as pl
from jax.experimental.pallas import tpu as pltpu
from jax.experimental.pallas import tpu_sc as plsc

D = 16
CH = 128
BLK = 6
PBLK = 4
NC = 2
NS = 16
NW = NC * NS


def _round_up(a: int, b: int) -> int:
    return (a + b - 1) // b * b


def _make_agg(n_nodes: int, cpw: int, nacc: int):
    mesh = plsc.VectorSubcoreMesh(core_axis_name="c", subcore_axis_name="s")
    zrows = nacc // NS
    m_blocks = cpw // BLK
    assert m_blocks >= 3

    def body(h_hbm, srcr_hbm, dstr_hbm, zeros_hbm, out_hbm,
             sidx, didx, rows, acc, gsem, ssem, isem):
        c = lax.axis_index("c")
        s = lax.axis_index("s")
        w = c * NS + s
        base = w * cpw

        pltpu.sync_copy(zeros_hbm.at[pl.ds(s * zrows, zrows)],
                        acc.at[pl.ds(s * zrows, zrows)])
        plsc.subcore_barrier()

        def idx_copy_sync(m, slot):
            cb = base + m * BLK
            pltpu.sync_copy(srcr_hbm.at[pl.ds(cb, BLK)],
                            sidx.at[pl.ds(slot * BLK, BLK)])
            pltpu.sync_copy(dstr_hbm.at[pl.ds(cb, BLK)],
                            didx.at[pl.ds(slot * BLK, BLK)])

        def idx_copy_async(m, slot):
            cb = base + m * BLK
            pltpu.async_copy(srcr_hbm.at[pl.ds(cb, BLK)],
                             sidx.at[pl.ds(slot * BLK, BLK)], isem)
            pltpu.async_copy(dstr_hbm.at[pl.ds(cb, BLK)],
                             didx.at[pl.ds(slot * BLK, BLK)], isem)

        def fire_gathers(slot_i, slot_r):
            for j in range(BLK):
                pltpu.async_copy(h_hbm.at[sidx.at[slot_i * BLK + j]],
                                 rows.at[pl.ds((slot_r * BLK + j) * CH, CH)],
                                 gsem)

        def fire_scatters(slot_i, slot_r):
            for j in range(BLK):
                pltpu.async_copy(rows.at[pl.ds((slot_r * BLK + j) * CH, CH)],
                                 acc.at[didx.at[slot_i * BLK + j]],
                                 ssem, add=True)

        def drain(sem, k):
            for _ in range(k):
                pltpu.make_async_copy(h_hbm.at[pl.ds(0, CH)],
                                      rows.at[pl.ds(0, CH)], sem).wait()

        def drain_idx():
            for ref in (sidx, didx):
                pltpu.make_async_copy(srcr_hbm.at[pl.ds(0, BLK)],
                                      ref.at[pl.ds(0, BLK)], isem).wait()

        idx_copy_sync(0, 0)
        fire_gathers(0, 0)
        idx_copy_sync(1, 1)

        def loop_body(m, carry):
            si = lax.rem(m, 3)
            sr = lax.rem(m, 2)
            sin_ = lax.rem(m + 1, 3)
            srn = lax.rem(m + 1, 2)
            sif = lax.rem(m + 2, 3)

            @pl.when(m >= 1)
            def _():
                drain(ssem, BLK)
                drain_idx()

            fire_gathers(sin_, srn)
            drain(gsem, BLK)
            fire_scatters(si, sr)

            @pl.when(m < m_blocks - 2)
            def _():
                idx_copy_async(m + 2, sif)

            return carry

        lax.fori_loop(0, m_blocks - 1, loop_body, 0)

        drain(ssem, BLK)
        drain(gsem, BLK)
        fire_scatters((m_blocks - 1) % 3, (m_blocks - 1) % 2)
        drain(ssem, BLK)
        plsc.subcore_barrier()

        pltpu.sync_copy(acc.at[pl.ds(s * zrows, zrows)],
                        out_hbm.at[c].at[pl.ds(s * zrows, zrows)])

    return pl.kernel(
        body,
        out_type=jax.ShapeDtypeStruct((NC, nacc, D), jnp.float32),
        mesh=mesh,
        scratch_types=[
            pltpu.VMEM((3 * BLK, CH), jnp.int32),
            pltpu.VMEM((3 * BLK, CH), jnp.int32),
            pltpu.VMEM((2 * BLK * CH, D), jnp.float32),
            pltpu.VMEM_SHARED((nacc, D), jnp.float32),
            pltpu.SemaphoreType.DMA,
            pltpu.SemaphoreType.DMA,
            pltpu.SemaphoreType.DMA,
        ],
        compiler_params=pltpu.CompilerParams(use_tc_tiling_on_sc=False),
    )


def _make_pick(pp: int):
    mesh = plsc.VectorSubcoreMesh(core_axis_name="c", subcore_axis_name="s")
    cpw = pp // (NW * CH)
    m_blocks = cpw // PBLK
    assert m_blocks >= 3

    def body(h_hbm, idxr_hbm, out_hbm, sidx, rows, gsem, isem):
        c = lax.axis_index("c")
        s = lax.axis_index("s")
        w = c * NS + s
        base = w * cpw

        def idx_copy_sync(m, slot):
            pltpu.sync_copy(idxr_hbm.at[pl.ds(base + m * PBLK, PBLK)],
                            sidx.at[pl.ds(slot * PBLK, PBLK)])

        def idx_copy_async(m, slot):
            pltpu.async_copy(idxr_hbm.at[pl.ds(base + m * PBLK, PBLK)],
                             sidx.at[pl.ds(slot * PBLK, PBLK)], isem)

        def fire_gathers(slot_i, slot_r):
            for j in range(PBLK):
                pltpu.async_copy(h_hbm.at[sidx.at[slot_i * PBLK + j]],
                                 rows.at[pl.ds((slot_r * PBLK + j) * CH, CH)],
                                 gsem)

        def drain(k):
            for _ in range(k):
                pltpu.make_async_copy(h_hbm.at[pl.ds(0, CH)],
                                      rows.at[pl.ds(0, CH)], gsem).wait()

        def drain_idx():
            pltpu.make_async_copy(idxr_hbm.at[pl.ds(0, PBLK)],
                                  sidx.at[pl.ds(0, PBLK)], isem).wait()

        def block_out(m, slot_r):
            pltpu.sync_copy(rows.at[pl.ds(slot_r * PBLK * CH, PBLK * CH)],
                            out_hbm.at[pl.ds((base + m * PBLK) * CH,
                                             PBLK * CH)])

        idx_copy_sync(0, 0)
        fire_gathers(0, 0)
        idx_copy_sync(1, 1)

        def loop_body(m, carry):
            sin_ = lax.rem(m + 1, 4)
            srn = lax.rem(m + 1, 2)
            sif = lax.rem(m + 2, 4)

            @pl.when(m >= 1)
            def _():
                drain_idx()

            fire_gathers(sin_, srn)
            drain(PBLK)
            block_out(m, lax.rem(m, 2))

            @pl.when(m < m_blocks - 2)
            def _():
                idx_copy_async(m + 2, sif)

            return carry

        lax.fori_loop(0, m_blocks - 1, loop_body, 0)
        drain(PBLK)
        block_out(m_blocks - 1, (m_blocks - 1) % 2)

    return pl.kernel(
        body,
        out_type=jax.ShapeDtypeStruct((pp, D), jnp.float32),
        mesh=mesh,
        scratch_types=[
            pltpu.VMEM((4 * PBLK, CH), jnp.int32),
            pltpu.VMEM((2 * PBLK * CH, D), jnp.float32),
            pltpu.SemaphoreType.DMA,
            pltpu.SemaphoreType.DMA,
        ],
        compiler_params=pltpu.CompilerParams(use_tc_tiling_on_sc=False),
    )


def _enc_body(z_ref, w_ref, c_ref, o_ref):
    o_ref[...] = (jnp.dot(z_ref[...], w_ref[...],
                          preferred_element_type=jnp.float32) + c_ref[...])


def _lin_body(p_ref, w_ref, c_ref, o_ref, *, mean, relu_on):
    a = p_ref[0] + p_ref[1]
    if mean:
        a = a / jnp.maximum(a[:, 10:11], 1.0)
    y = jnp.dot(a, w_ref[...], preferred_element_type=jnp.float32) + c_ref[...]
    if relu_on:
        y = jnp.maximum(y, 0.0)
    o_ref[...] = y


def _dec_body(p_ref, w_ref, c_ref, o_ref):
    y = (jnp.dot(p_ref[...], w_ref[...],
                 preferred_element_type=jnp.float32) + c_ref[...])
    m = jnp.max(y, axis=1, keepdims=True)
    e = jnp.exp(y - m)
    o_ref[...] = (e / jnp.sum(e, axis=1, keepdims=True))[:, :10]


def _pad_w(w):
    wp = jnp.zeros((D, D), jnp.float32)
    return wp.at[:w.shape[0], :w.shape[1]].set(w)


def _cvec(b, ones_lane=True):
    c = jnp.zeros((1, D), jnp.float32).at[0, :b.shape[0]].set(b)
    if ones_lane:
        c = c.at[0, 10].set(1.0)
    return c


def kernel(x, z, edge_index, z1edge_index, z2edge_index, z3edge_index,
           edge_attr, pickable, W_enc, b_enc, W_c1, b_c1, W_c2, b_c2,
           W_x1, b_x1, W_lin, b_lin):
    n, zf = z.shape
    e = z1edge_index.shape[1]
    p = pickable.shape[0]
    assert n % NS == 0

    ep = _round_up(e, NW * CH * BLK)
    cpw = ep // (NW * CH)
    src = z1edge_index[0]
    dst = z1edge_index[1]
    if ep != e:
        src = jnp.concatenate([src, jnp.zeros((ep - e,), jnp.int32)])
        dst = jnp.concatenate([dst, jnp.full((ep - e,), n, jnp.int32)])
    srcr = src.reshape(ep // CH, CH)
    dstr = dst.reshape(ep // CH, CH)

    nacc = _round_up(n + 1, NS * 8)
    zeros_acc = jnp.zeros((nacc, D), jnp.float32)

    wenc = jnp.zeros((zf, D), jnp.float32).at[:, :W_enc.shape[1]].set(W_enc)
    cenc = _cvec(b_enc)
    w1, c1 = _pad_w(W_c1), _cvec(b_c1)
    w2, c2 = _pad_w(W_c2), _cvec(b_c2)
    wx, cx = _pad_w(W_x1), _cvec(b_x1)
    wl = _pad_w(W_lin)
    cl = jnp.full((1, D), -1e30, jnp.float32).at[0, :b_lin.shape[0]].set(b_lin)

    br = 2000
    nblk = n // br
    f32 = jnp.float32

    enc = pl.pallas_call(
        _enc_body,
        grid=(nblk,),
        in_specs=[pl.BlockSpec((br, zf), lambda i: (i, 0)),
                  pl.BlockSpec((zf, D), lambda i: (0, 0)),
                  pl.BlockSpec((1, D), lambda i: (0, 0))],
        out_specs=pl.BlockSpec((br, D), lambda i: (i, 0)),
        out_shape=jax.ShapeDtypeStruct((n, D), f32),
    )

    def lin(mean, relu_on):
        return pl.pallas_call(
            functools.partial(_lin_body, mean=mean, relu_on=relu_on),
            grid=(nblk,),
            in_specs=[pl.BlockSpec((NC, br, D), lambda i: (0, i, 0)),
                      pl.BlockSpec((D, D), lambda i: (0, 0)),
                      pl.BlockSpec((1, D), lambda i: (0, 0))],
            out_specs=pl.BlockSpec((br, D), lambda i: (i, 0)),
            out_shape=jax.ShapeDtypeStruct((n, D), f32),
        )

    agg = _make_agg(n, cpw, nacc)

    h0 = enc(z, wenc, cenc)
    h1 = lin(False, True)(agg(h0, srcr, dstr, zeros_acc), w1, c1)
    h2 = lin(False, False)(agg(h1, srcr, dstr, zeros_acc), w2, c2)
    h3 = lin(True, True)(agg(h2, srcr, dstr, zeros_acc), wx, cx)

    pp = _round_up(p, NW * CH * PBLK)
    pidx = pickable
    if pp != p:
        pidx = jnp.concatenate([pidx, jnp.zeros((pp - p,), jnp.int32)])
    pidxr = pidx.reshape(pp // CH, CH)
    picked = _make_pick(pp)(h3, pidxr)

    pbr = 2000
    pblk = p // pbr
    out = pl.pallas_call(
        _dec_body,
        grid=(pblk,),
        in_specs=[pl.BlockSpec((pbr, D), lambda i: (i, 0)),
                  pl.BlockSpec((D, D), lambda i: (0, 0)),
                  pl.BlockSpec((1, D), lambda i: (0, 0))],
        out_specs=pl.BlockSpec((pbr, 10), lambda i: (i, 0)),
        out_shape=jax.ShapeDtypeStruct((p, 10), f32),
    )(picked, wl, cl)
    return out


# speedup vs baseline: 46.8647x; 1.3653x over previous
import functools

import jax
import jax.numpy as jnp
from jax import lax
from jax.experimental import pallas as pl
from jax.experimental.pallas import tpu as pltpu
from jax.experimental.pallas import tpu_sc as plsc

D = 16
CH = 128
BLK = 6
PBLK = 4
NC = 2
NS = 16
NW = NC * NS


def _round_up(a: int, b: int) -> int:
    return (a + b - 1) // b * b


def _make_agg(n_nodes: int, cpw: int, nacc: int):
    mesh = plsc.VectorSubcoreMesh(core_axis_name="c", subcore_axis_name="s")
    zrows = nacc // NS
    m_blocks = cpw // BLK
    assert m_blocks >= 3

    def body(h_hbm, srcr_hbm, dstr_hbm, zeros_hbm, out_hbm,
             sidx, didx, rows, acc, gsem, ssem, isem):
        c = lax.axis_index("c")
        s = lax.axis_index("s")
        w = c * NS + s
        base = w * cpw

        pltpu.sync_copy(zeros_hbm.at[pl.ds(s * zrows, zrows)],
                        acc.at[pl.ds(s * zrows, zrows)])
        plsc.subcore_barrier()

        def idx_copy_sync(m, slot):
            cb = base + m * BLK
            pltpu.sync_copy(srcr_hbm.at[pl.ds(cb, BLK)],
                            sidx.at[pl.ds(slot * BLK, BLK)])
            pltpu.sync_copy(dstr_hbm.at[pl.ds(cb, BLK)],
                            didx.at[pl.ds(slot * BLK, BLK)])

        def idx_copy_async(m, slot):
            cb = base + m * BLK
            pltpu.async_copy(srcr_hbm.at[pl.ds(cb, BLK)],
                             sidx.at[pl.ds(slot * BLK, BLK)], isem)
            pltpu.async_copy(dstr_hbm.at[pl.ds(cb, BLK)],
                             didx.at[pl.ds(slot * BLK, BLK)], isem)

        def fire_gathers(slot_i, slot_r):
            for j in range(BLK):
                pltpu.async_copy(h_hbm.at[sidx.at[slot_i * BLK + j]],
                                 rows.at[pl.ds((slot_r * BLK + j) * CH, CH)],
                                 gsem)

        def fire_scatters(slot_i, slot_r):
            for j in range(BLK):
                pltpu.async_copy(rows.at[pl.ds((slot_r * BLK + j) * CH, CH)],
                                 acc.at[didx.at[slot_i * BLK + j]],
                                 ssem, add=True)

        def drain(sem, k):
            for _ in range(k):
                pltpu.make_async_copy(h_hbm.at[pl.ds(0, CH)],
                                      rows.at[pl.ds(0, CH)], sem).wait()

        def drain_idx():
            for ref in (sidx, didx):
                pltpu.make_async_copy(srcr_hbm.at[pl.ds(0, BLK)],
                                      ref.at[pl.ds(0, BLK)], isem).wait()

        idx_copy_sync(0, 0)
        fire_gathers(0, 0)
        idx_copy_sync(1, 1)

        def loop_body(m, carry):
            si = lax.rem(m, 3)
            sr = lax.rem(m, 2)
            sin_ = lax.rem(m + 1, 3)
            srn = lax.rem(m + 1, 2)
            sif = lax.rem(m + 2, 3)

            @pl.when(m >= 1)
            def _():
                drain(ssem, BLK)
                drain_idx()

            fire_gathers(sin_, srn)
            drain(gsem, BLK)
            fire_scatters(si, sr)

            @pl.when(m < m_blocks - 2)
            def _():
                idx_copy_async(m + 2, sif)

            return carry

        lax.fori_loop(0, m_blocks - 1, loop_body, 0)

        drain(ssem, BLK)
        drain(gsem, BLK)
        fire_scatters((m_blocks - 1) % 3, (m_blocks - 1) % 2)
        drain(ssem, BLK)
        plsc.subcore_barrier()

        pltpu.sync_copy(acc.at[pl.ds(s * zrows, zrows)],
                        out_hbm.at[c].at[pl.ds(s * zrows, zrows)])

    return pl.kernel(
        body,
        out_type=jax.ShapeDtypeStruct((NC, nacc, D), jnp.float32),
        mesh=mesh,
        scratch_types=[
            pltpu.VMEM((3 * BLK, CH), jnp.int32),
            pltpu.VMEM((3 * BLK, CH), jnp.int32),
            pltpu.VMEM((2 * BLK * CH, D), jnp.float32),
            pltpu.VMEM_SHARED((nacc, D), jnp.float32),
            pltpu.SemaphoreType.DMA,
            pltpu.SemaphoreType.DMA,
            pltpu.SemaphoreType.DMA,
        ],
        compiler_params=pltpu.CompilerParams(use_tc_tiling_on_sc=False),
    )


def _make_pick(pp: int):
    mesh = plsc.VectorSubcoreMesh(core_axis_name="c", subcore_axis_name="s")
    cpw = pp // (NW * CH)
    m_blocks = cpw // PBLK
    assert m_blocks >= 3

    def body(h_hbm, idxr_hbm, out_hbm, sidx, rows, gsem, isem):
        c = lax.axis_index("c")
        s = lax.axis_index("s")
        w = c * NS + s
        base = w * cpw

        def idx_copy_sync(m, slot):
            pltpu.sync_copy(idxr_hbm.at[pl.ds(base + m * PBLK, PBLK)],
                            sidx.at[pl.ds(slot * PBLK, PBLK)])

        def idx_copy_async(m, slot):
            pltpu.async_copy(idxr_hbm.at[pl.ds(base + m * PBLK, PBLK)],
                             sidx.at[pl.ds(slot * PBLK, PBLK)], isem)

        def fire_gathers(slot_i, slot_r):
            for j in range(PBLK):
                pltpu.async_copy(h_hbm.at[sidx.at[slot_i * PBLK + j]],
                                 rows.at[pl.ds((slot_r * PBLK + j) * CH, CH)],
                                 gsem)

        def drain(k):
            for _ in range(k):
                pltpu.make_async_copy(h_hbm.at[pl.ds(0, CH)],
                                      rows.at[pl.ds(0, CH)], gsem).wait()

        def drain_idx():
            pltpu.make_async_copy(idxr_hbm.at[pl.ds(0, PBLK)],
                                  sidx.at[pl.ds(0, PBLK)], isem).wait()

        def block_out(m, slot_r):
            pltpu.sync_copy(rows.at[pl.ds(slot_r * PBLK * CH, PBLK * CH)],
                            out_hbm.at[pl.ds((base + m * PBLK) * CH,
                                             PBLK * CH)])

        idx_copy_sync(0, 0)
        fire_gathers(0, 0)
        idx_copy_sync(1, 1)

        def loop_body(m, carry):
            sin_ = lax.rem(m + 1, 4)
            srn = lax.rem(m + 1, 2)
            sif = lax.rem(m + 2, 4)

            @pl.when(m >= 1)
            def _():
                drain_idx()

            fire_gathers(sin_, srn)
            drain(PBLK)
            block_out(m, lax.rem(m, 2))

            @pl.when(m < m_blocks - 2)
            def _():
                idx_copy_async(m + 2, sif)

            return carry

        lax.fori_loop(0, m_blocks - 1, loop_body, 0)
        drain(PBLK)
        block_out(m_blocks - 1, (m_blocks - 1) % 2)

    return pl.kernel(
        body,
        out_type=jax.ShapeDtypeStruct((pp, D), jnp.float32),
        mesh=mesh,
        scratch_types=[
            pltpu.VMEM((4 * PBLK, CH), jnp.int32),
            pltpu.VMEM((2 * PBLK * CH, D), jnp.float32),
            pltpu.SemaphoreType.DMA,
            pltpu.SemaphoreType.DMA,
        ],
        compiler_params=pltpu.CompilerParams(use_tc_tiling_on_sc=False),
    )


def _enc_body(z_ref, w_ref, c_ref, o_ref):
    acc = c_ref[...]
    for k in range(8):
        acc = acc + jnp.dot(z_ref[:, k, :], w_ref[k],
                            preferred_element_type=jnp.float32)
    o_ref[...] = acc[:, None, :]


def _lin_body(p_ref, w_ref, s_ref, c_ref, o_ref, *, mean, relu_on):
    a = p_ref[0] + p_ref[1]
    if mean:
        deg = jnp.dot(a, s_ref[...], preferred_element_type=jnp.float32)
        a = a / jnp.maximum(deg, 1.0)
    y = jnp.dot(a, w_ref[...], preferred_element_type=jnp.float32) + c_ref[...]
    if relu_on:
        y = jnp.maximum(y, 0.0)
    o_ref[...] = y


def _dec_body(p_ref, w_ref, c_ref, o_ref):
    y = (jnp.dot(p_ref[...], w_ref[...],
                 preferred_element_type=jnp.float32) + c_ref[...])
    parts = []
    for k in range(8):
        yk = y[:, 16 * k:16 * (k + 1)]
        m = jnp.max(yk, axis=1, keepdims=True)
        e = jnp.exp(yk - m)
        parts.append(e / jnp.sum(e, axis=1, keepdims=True))
    o_ref[...] = jnp.concatenate(parts, axis=1)


def _pad_w(w):
    wp = jnp.zeros((D, D), jnp.float32)
    return wp.at[:w.shape[0], :w.shape[1]].set(w)


def _cvec(b, ones_lane=True):
    c = jnp.zeros((1, D), jnp.float32).at[0, :b.shape[0]].set(b)
    if ones_lane:
        c = c.at[0, 10].set(1.0)
    return c


def kernel(x, z, edge_index, z1edge_index, z2edge_index, z3edge_index,
           edge_attr, pickable, W_enc, b_enc, W_c1, b_c1, W_c2, b_c2,
           W_x1, b_x1, W_lin, b_lin):
    n, zf = z.shape
    e = z1edge_index.shape[1]
    p = pickable.shape[0]
    assert n % 8 == 0

    nacc = _round_up(n + 1, NS * 8 * 8)
    nrp = nacc * D // 128
    f32 = jnp.float32

    ep = _round_up(e, NW * CH * BLK)
    cpw = ep // (NW * CH)
    src = z1edge_index[0]
    dst = z1edge_index[1]
    if ep != e:
        src = jnp.concatenate([src, jnp.zeros((ep - e,), jnp.int32)])
        dst = jnp.concatenate([dst, jnp.full((ep - e,), n, jnp.int32)])
    srcr = src.reshape(ep // CH, CH)
    dstr = dst.reshape(ep // CH, CH)

    zeros_acc = jnp.zeros((nacc, D), f32)

    wencp = jnp.zeros((zf, D), f32).at[:, :W_enc.shape[1]].set(W_enc)
    wenc8 = jnp.zeros((8, zf, 128), f32)
    for k in range(8):
        wenc8 = wenc8.at[k, :, D * k:D * (k + 1)].set(wencp)
    eye8 = jnp.eye(8, dtype=f32)

    def w128(w):
        return jnp.kron(eye8, _pad_w(w))

    def c128(b, ones_lane=True):
        return jnp.tile(_cvec(b, ones_lane), (1, 8))

    cenc = c128(b_enc)
    w1, c1 = w128(W_c1), c128(b_c1)
    w2, c2 = w128(W_c2), c128(b_c2)
    wx, cx = w128(W_x1), c128(b_x1)
    wl = w128(W_lin)
    clv = jnp.full((1, D), -1e30, f32).at[0, :b_lin.shape[0]].set(b_lin)
    cl = jnp.tile(clv, (1, 8))
    s128 = jnp.zeros((128, 128), f32)
    for k in range(8):
        s128 = s128.at[D * k + 10, D * k:D * (k + 1)].set(1.0)

    grp = nrp // 4
    z3 = z.reshape(n // 8, 8, zf)
    enc_bl = nrp // 8

    h0p = pl.pallas_call(
        _enc_body,
        grid=(8,),
        in_specs=[pl.BlockSpec((enc_bl, 8, zf), lambda i: (i, 0, 0)),
                  pl.BlockSpec((8, zf, 128), lambda i: (0, 0, 0)),
                  pl.BlockSpec((1, 128), lambda i: (0, 0))],
        out_specs=pl.BlockSpec((enc_bl, 1, 128), lambda i: (i, 0, 0)),
        out_shape=jax.ShapeDtypeStruct((nrp, 1, 128), f32),
    )(z3, wenc8, cenc)
    h0 = h0p.reshape(nacc, D)

    def lin(mean, relu_on):
        return pl.pallas_call(
            functools.partial(_lin_body, mean=mean, relu_on=relu_on),
            grid=(4,),
            in_specs=[pl.BlockSpec((NC, grp, 128), lambda i: (0, i, 0)),
                      pl.BlockSpec((128, 128), lambda i: (0, 0)),
                      pl.BlockSpec((128, 128), lambda i: (0, 0)),
                      pl.BlockSpec((1, 128), lambda i: (0, 0))],
            out_specs=pl.BlockSpec((grp, 128), lambda i: (i, 0)),
            out_shape=jax.ShapeDtypeStruct((nrp, 128), f32),
        )

    agg = _make_agg(n, cpw, nacc)

    def layer(h, mean, relu_on, w, cv):
        prt = agg(h, srcr, dstr, zeros_acc)
        prtp = prt.reshape(NC, nrp, 128)
        return lin(mean, relu_on)(prtp, w, s128, cv).reshape(nacc, D)

    h1 = layer(h0, False, True, w1, c1)
    h2 = layer(h1, False, False, w2, c2)
    h3 = layer(h2, True, True, wx, cx)

    pp = _round_up(p, NW * CH * PBLK)
    pidx = pickable
    if pp != p:
        pidx = jnp.concatenate([pidx, jnp.zeros((pp - p,), jnp.int32)])
    pidxr = pidx.reshape(pp // CH, CH)
    picked = _make_pick(pp)(h3, pidxr)

    prp = pp * D // 128
    dec_bl = prp // 8
    outp = pl.pallas_call(
        _dec_body,
        grid=(8,),
        in_specs=[pl.BlockSpec((dec_bl, 128), lambda i: (i, 0)),
                  pl.BlockSpec((128, 128), lambda i: (0, 0)),
                  pl.BlockSpec((1, 128), lambda i: (0, 0))],
        out_specs=pl.BlockSpec((dec_bl, 128), lambda i: (i, 0)),
        out_shape=jax.ShapeDtypeStruct((prp, 128), f32),
    )(picked.reshape(prp, 128), wl, cl)
    return outp.reshape(pp, D)[:p, :10]


# speedup vs baseline: 57.3875x; 1.2245x over previous
import functools

import jax
import jax.numpy as jnp
from jax import lax
from jax.experimental import pallas as pl
from jax.experimental.pallas import tpu as pltpu
from jax.experimental.pallas import tpu_sc as plsc

D = 16
CH = 128
BLK = 6
PBLK = 4
NC = 2
NS = 16
NW = NC * NS
ZCH = 512
F0_NUM, F0_DEN = 39, 100


def _round_up(a: int, b: int) -> int:
    return (a + b - 1) // b * b


def _make_agg(tch: int, nacc: int):
    mesh = plsc.VectorSubcoreMesh(core_axis_name="c", subcore_axis_name="s")
    zrows = nacc // NS
    ch0 = tch * F0_NUM // F0_DEN
    ch1 = tch - ch0
    assert min(ch0, ch1) // NS >= 3 * BLK

    def body(h_hbm, srcr_hbm, dstr_hbm, out_hbm,
             sidx, didx, rows, acc, gsem, ssem, isem):
        c = lax.axis_index("c")
        s = lax.axis_index("s")

        zvec = jnp.zeros((D,), jnp.float32)

        def zfill(i, carry):
            rows[i] = zvec
            return carry

        lax.fori_loop(0, ZCH, zfill, 0)
        zbase = s * zrows
        nfull = zrows // ZCH
        for k in range(nfull):
            pltpu.sync_copy(rows.at[pl.ds(0, ZCH)],
                            acc.at[pl.ds(zbase + k * ZCH, ZCH)])
        ztail = zrows - nfull * ZCH
        if ztail:
            pltpu.sync_copy(rows.at[pl.ds(0, ztail)],
                            acc.at[pl.ds(zbase + nfull * ZCH, ztail)])
        plsc.subcore_barrier()

        cch = jnp.where(c == 0, ch0, ch1)
        cbase = c * ch0
        lo = cbase + s * cch // NS
        hi = cbase + (s + 1) * cch // NS
        m_full = (hi - lo) // BLK
        tail = (hi - lo) - m_full * BLK

        def idx_copy_sync(m, slot):
            cb = lo + m * BLK
            pltpu.sync_copy(srcr_hbm.at[pl.ds(cb, BLK)],
                            sidx.at[pl.ds(slot * BLK, BLK)])
            pltpu.sync_copy(dstr_hbm.at[pl.ds(cb, BLK)],
                            didx.at[pl.ds(slot * BLK, BLK)])

        def idx_copy_async(m, slot):
            cb = lo + m * BLK
            pltpu.async_copy(srcr_hbm.at[pl.ds(cb, BLK)],
                             sidx.at[pl.ds(slot * BLK, BLK)], isem)
            pltpu.async_copy(dstr_hbm.at[pl.ds(cb, BLK)],
                             didx.at[pl.ds(slot * BLK, BLK)], isem)

        def fire_gathers(slot_i, slot_r):
            for j in range(BLK):
                pltpu.async_copy(h_hbm.at[sidx.at[slot_i * BLK + j]],
                                 rows.at[pl.ds((slot_r * BLK + j) * CH, CH)],
                                 gsem)

        def fire_scatters(slot_i, slot_r):
            for j in range(BLK):
                pltpu.async_copy(rows.at[pl.ds((slot_r * BLK + j) * CH, CH)],
                                 acc.at[didx.at[slot_i * BLK + j]],
                                 ssem, add=True)

        def drain(sem, k):
            for _ in range(k):
                pltpu.make_async_copy(h_hbm.at[pl.ds(0, CH)],
                                      rows.at[pl.ds(0, CH)], sem).wait()

        def drain_idx():
            for ref in (sidx, didx):
                pltpu.make_async_copy(srcr_hbm.at[pl.ds(0, BLK)],
                                      ref.at[pl.ds(0, BLK)], isem).wait()

        idx_copy_sync(0, 0)
        fire_gathers(0, 0)
        idx_copy_sync(1, 1)

        def loop_body(m, carry):
            si = lax.rem(m, 3)
            sr = lax.rem(m, 2)
            sin_ = lax.rem(m + 1, 3)
            srn = lax.rem(m + 1, 2)
            sif = lax.rem(m + 2, 3)

            @pl.when(m >= 1)
            def _():
                drain(ssem, BLK)
                drain_idx()

            fire_gathers(sin_, srn)
            drain(gsem, BLK)
            fire_scatters(si, sr)

            @pl.when(m < m_full - 2)
            def _():
                idx_copy_async(m + 2, sif)

            return carry

        lax.fori_loop(0, m_full - 1, loop_body, 0)

        drain(ssem, BLK)
        drain(gsem, BLK)
        fire_scatters(lax.rem(m_full - 1, 3), lax.rem(m_full - 1, 2))
        drain(ssem, BLK)

        def tail_body(t, carry):
            cb = lo + m_full * BLK + t
            pltpu.sync_copy(srcr_hbm.at[pl.ds(cb, 1)], sidx.at[pl.ds(0, 1)])
            pltpu.sync_copy(dstr_hbm.at[pl.ds(cb, 1)], didx.at[pl.ds(0, 1)])
            pltpu.async_copy(h_hbm.at[sidx.at[0]],
                             rows.at[pl.ds(0, CH)], gsem).wait()
            pltpu.async_copy(rows.at[pl.ds(0, CH)],
                             acc.at[didx.at[0]], ssem, add=True).wait()
            return carry

        lax.fori_loop(0, tail, tail_body, 0)
        plsc.subcore_barrier()

        pltpu.sync_copy(acc.at[pl.ds(s * zrows, zrows)],
                        out_hbm.at[c].at[pl.ds(s * zrows, zrows)])

    return pl.kernel(
        body,
        out_type=jax.ShapeDtypeStruct((NC, nacc, D), jnp.float32),
        mesh=mesh,
        scratch_types=[
            pltpu.VMEM((3 * BLK, CH), jnp.int32),
            pltpu.VMEM((3 * BLK, CH), jnp.int32),
            pltpu.VMEM((2 * BLK * CH, D), jnp.float32),
            pltpu.VMEM_SHARED((nacc, D), jnp.float32),
            pltpu.SemaphoreType.DMA,
            pltpu.SemaphoreType.DMA,
            pltpu.SemaphoreType.DMA,
        ],
        compiler_params=pltpu.CompilerParams(use_tc_tiling_on_sc=False),
    )


def _make_pick(pp: int):
    mesh = plsc.VectorSubcoreMesh(core_axis_name="c", subcore_axis_name="s")
    cpw = pp // (NW * CH)
    m_blocks = cpw // PBLK
    assert m_blocks >= 3

    def body(h_hbm, idxr_hbm, out_hbm, sidx, rows, gsem, isem):
        c = lax.axis_index("c")
        s = lax.axis_index("s")
        w = c * NS + s
        base = w * cpw

        def idx_copy_sync(m, slot):
            pltpu.sync_copy(idxr_hbm.at[pl.ds(base + m * PBLK, PBLK)],
                            sidx.at[pl.ds(slot * PBLK, PBLK)])

        def idx_copy_async(m, slot):
            pltpu.async_copy(idxr_hbm.at[pl.ds(base + m * PBLK, PBLK)],
                             sidx.at[pl.ds(slot * PBLK, PBLK)], isem)

        def fire_gathers(slot_i, slot_r):
            for j in range(PBLK):
                pltpu.async_copy(h_hbm.at[sidx.at[slot_i * PBLK + j]],
                                 rows.at[pl.ds((slot_r * PBLK + j) * CH, CH)],
                                 gsem)

        def drain(k):
            for _ in range(k):
                pltpu.make_async_copy(h_hbm.at[pl.ds(0, CH)],
                                      rows.at[pl.ds(0, CH)], gsem).wait()

        def drain_idx():
            pltpu.make_async_copy(idxr_hbm.at[pl.ds(0, PBLK)],
                                  sidx.at[pl.ds(0, PBLK)], isem).wait()

        def block_out(m, slot_r):
            pltpu.sync_copy(rows.at[pl.ds(slot_r * PBLK * CH, PBLK * CH)],
                            out_hbm.at[pl.ds((base + m * PBLK) * CH,
                                             PBLK * CH)])

        idx_copy_sync(0, 0)
        fire_gathers(0, 0)
        idx_copy_sync(1, 1)

        def loop_body(m, carry):
            sin_ = lax.rem(m + 1, 4)
            srn = lax.rem(m + 1, 2)
            sif = lax.rem(m + 2, 4)

            @pl.when(m >= 1)
            def _():
                drain_idx()

            fire_gathers(sin_, srn)
            drain(PBLK)
            block_out(m, lax.rem(m, 2))

            @pl.when(m < m_blocks - 2)
            def _():
                idx_copy_async(m + 2, sif)

            return carry

        lax.fori_loop(0, m_blocks - 1, loop_body, 0)
        drain(PBLK)
        block_out(m_blocks - 1, (m_blocks - 1) % 2)

    return pl.kernel(
        body,
        out_type=jax.ShapeDtypeStruct((pp, D), jnp.float32),
        mesh=mesh,
        scratch_types=[
            pltpu.VMEM((4 * PBLK, CH), jnp.int32),
            pltpu.VMEM((2 * PBLK * CH, D), jnp.float32),
            pltpu.SemaphoreType.DMA,
            pltpu.SemaphoreType.DMA,
        ],
        compiler_params=pltpu.CompilerParams(use_tc_tiling_on_sc=False),
    )


def _enc_body(z_ref, w_ref, c_ref, o_ref):
    acc = c_ref[...]
    for k in range(8):
        acc = acc + jnp.dot(z_ref[:, k, :], w_ref[k],
                            preferred_element_type=jnp.float32)
    o_ref[...] = acc[:, None, :]


def _lin_body(p_ref, w_ref, s_ref, c_ref, o_ref, *, mean, relu_on):
    a = p_ref[0] + p_ref[1]
    if mean:
        deg = jnp.dot(a, s_ref[...], preferred_element_type=jnp.float32)
        a = a / jnp.maximum(deg, 1.0)
    y = jnp.dot(a, w_ref[...], preferred_element_type=jnp.float32) + c_ref[...]
    if relu_on:
        y = jnp.maximum(y, 0.0)
    o_ref[...] = y


def _dec_body(p_ref, w_ref, c_ref, o_ref):
    y = (jnp.dot(p_ref[...], w_ref[...],
                 preferred_element_type=jnp.float32) + c_ref[...])
    rows = o_ref.shape[0]
    for k in range(8):
        yk = y[:, 16 * k:16 * (k + 1)]
        m = jnp.max(yk, axis=1, keepdims=True)
        e = jnp.exp(yk - m)
        pk = e / jnp.sum(e, axis=1, keepdims=True)
        o_ref[:, k, :] = pk[:rows, :10]


def _pad_w(w):
    wp = jnp.zeros((D, D), jnp.float32)
    return wp.at[:w.shape[0], :w.shape[1]].set(w)


def _cvec(b, ones_lane=True):
    c = jnp.zeros((1, D), jnp.float32).at[0, :b.shape[0]].set(b)
    if ones_lane:
        c = c.at[0, 10].set(1.0)
    return c


def kernel(x, z, edge_index, z1edge_index, z2edge_index, z3edge_index,
           edge_attr, pickable, W_enc, b_enc, W_c1, b_c1, W_c2, b_c2,
           W_x1, b_x1, W_lin, b_lin):
    n, zf = z.shape
    e = z1edge_index.shape[1]
    p = pickable.shape[0]
    assert n % 8 == 0 and e % CH == 0

    nacc = _round_up(n, 256)
    nrp = nacc * D // 128
    f32 = jnp.float32

    tch = e // CH
    srcr = z1edge_index[0].reshape(tch, CH)
    dstr = z1edge_index[1].reshape(tch, CH)

    wencp = jnp.zeros((zf, D), f32).at[:, :W_enc.shape[1]].set(W_enc)
    wenc8 = jnp.zeros((8, zf, 128), f32)
    for k in range(8):
        wenc8 = wenc8.at[k, :, D * k:D * (k + 1)].set(wencp)
    eye8 = jnp.eye(8, dtype=f32)

    def w128(w):
        return jnp.kron(eye8, _pad_w(w))

    def c128(b, ones_lane=True):
        return jnp.tile(_cvec(b, ones_lane), (1, 8))

    cenc = c128(b_enc)
    w1, c1 = w128(W_c1), c128(b_c1)
    w2, c2 = w128(W_c2), c128(b_c2)
    wx, cx = w128(W_x1), c128(b_x1)
    wl = w128(W_lin)
    clv = jnp.full((1, D), -1e30, f32).at[0, :b_lin.shape[0]].set(b_lin)
    cl = jnp.tile(clv, (1, 8))
    s128 = jnp.zeros((128, 128), f32)
    for k in range(8):
        s128 = s128.at[D * k + 10, D * k:D * (k + 1)].set(1.0)

    grp = nrp // 4
    z3 = z.reshape(n // 8, 8, zf)
    enc_bl = nrp // 8

    h0p = pl.pallas_call(
        _enc_body,
        grid=(8,),
        in_specs=[pl.BlockSpec((enc_bl, 8, zf), lambda i: (i, 0, 0)),
                  pl.BlockSpec((8, zf, 128), lambda i: (0, 0, 0)),
                  pl.BlockSpec((1, 128), lambda i: (0, 0))],
        out_specs=pl.BlockSpec((enc_bl, 1, 128), lambda i: (i, 0, 0)),
        out_shape=jax.ShapeDtypeStruct((nrp, 1, 128), f32),
    )(z3, wenc8, cenc)
    h0 = h0p.reshape(nacc, D)

    def lin(mean, relu_on):
        return pl.pallas_call(
            functools.partial(_lin_body, mean=mean, relu_on=relu_on),
            grid=(4,),
            in_specs=[pl.BlockSpec((NC, grp, 128), lambda i: (0, i, 0)),
                      pl.BlockSpec((128, 128), lambda i: (0, 0)),
                      pl.BlockSpec((128, 128), lambda i: (0, 0)),
                      pl.BlockSpec((1, 128), lambda i: (0, 0))],
            out_specs=pl.BlockSpec((grp, 128), lambda i: (i, 0)),
            out_shape=jax.ShapeDtypeStruct((nrp, 128), f32),
        )

    agg = _make_agg(tch, nacc)

    def layer(h, mean, relu_on, w, cv):
        prt = agg(h, srcr, dstr)
        prtp = prt.reshape(NC, nrp, 128)
        return lin(mean, relu_on)(prtp, w, s128, cv).reshape(nacc, D)

    h1 = layer(h0, False, True, w1, c1)
    h2 = layer(h1, False, False, w2, c2)
    h3 = layer(h2, True, True, wx, cx)

    pp = _round_up(p, NW * CH * PBLK)
    pidx = pickable
    if pp != p:
        pidx = jnp.concatenate([pidx, jnp.zeros((pp - p,), jnp.int32)])
    pidxr = pidx.reshape(pp // CH, CH)
    picked = _make_pick(pp)(h3, pidxr)

    prp = pp * D // 128
    out3 = pl.pallas_call(
        _dec_body,
        grid=(1,),
        in_specs=[pl.BlockSpec((prp, 128), lambda i: (0, 0, 0)[:2]),
                  pl.BlockSpec((128, 128), lambda i: (0, 0)),
                  pl.BlockSpec((1, 128), lambda i: (0, 0))],
        out_specs=pl.BlockSpec((p // 8, 8, 10), lambda i: (0, 0, 0)),
        out_shape=jax.ShapeDtypeStruct((p // 8, 8, 10), f32),
    )(picked.reshape(prp, 128), wl, cl)
    return out3.reshape(p, 10)


# speedup vs baseline: 57.4155x; 1.0005x over previous
import functools

import jax
import jax.numpy as jnp
from jax import lax
from jax.experimental import pallas as pl
from jax.experimental.pallas import tpu as pltpu
from jax.experimental.pallas import tpu_sc as plsc

D = 16
CH = 128
BLK = 6
PBLK = 4
NC = 2
NS = 16
NW = NC * NS
ZCH = 512
F0_NUM, F0_DEN = 30, 100


def _round_up(a: int, b: int) -> int:
    return (a + b - 1) // b * b


def _make_agg(tch: int, nacc: int):
    mesh = plsc.VectorSubcoreMesh(core_axis_name="c", subcore_axis_name="s")
    zrows = nacc // NS
    ch0 = tch * F0_NUM // F0_DEN
    ch1 = tch - ch0
    assert min(ch0, ch1) // NS >= 3 * BLK

    def body(h_hbm, srcr_hbm, dstr_hbm, out_hbm,
             sidx, didx, rows, acc, gsem, ssem, isem):
        c = lax.axis_index("c")
        s = lax.axis_index("s")

        zvec = jnp.zeros((D,), jnp.float32)

        def zfill(i, carry):
            rows[i] = zvec
            return carry

        lax.fori_loop(0, ZCH, zfill, 0)
        zbase = s * zrows
        nfull = zrows // ZCH
        for k in range(nfull):
            pltpu.sync_copy(rows.at[pl.ds(0, ZCH)],
                            acc.at[pl.ds(zbase + k * ZCH, ZCH)])
        ztail = zrows - nfull * ZCH
        if ztail:
            pltpu.sync_copy(rows.at[pl.ds(0, ztail)],
                            acc.at[pl.ds(zbase + nfull * ZCH, ztail)])
        plsc.subcore_barrier()

        cch = jnp.where(c == 0, ch0, ch1)
        cbase = c * ch0
        lo = cbase + s * cch // NS
        hi = cbase + (s + 1) * cch // NS
        m_full = (hi - lo) // BLK
        tail = (hi - lo) - m_full * BLK

        def idx_copy_sync(m, slot):
            cb = lo + m * BLK
            pltpu.sync_copy(srcr_hbm.at[pl.ds(cb, BLK)],
                            sidx.at[pl.ds(slot * BLK, BLK)])
            pltpu.sync_copy(dstr_hbm.at[pl.ds(cb, BLK)],
                            didx.at[pl.ds(slot * BLK, BLK)])

        def idx_copy_async(m, slot):
            cb = lo + m * BLK
            pltpu.async_copy(srcr_hbm.at[pl.ds(cb, BLK)],
                             sidx.at[pl.ds(slot * BLK, BLK)], isem)
            pltpu.async_copy(dstr_hbm.at[pl.ds(cb, BLK)],
                             didx.at[pl.ds(slot * BLK, BLK)], isem)

        def fire_gathers(slot_i, slot_r):
            for j in range(BLK):
                pltpu.async_copy(h_hbm.at[sidx.at[slot_i * BLK + j]],
                                 rows.at[pl.ds((slot_r * BLK + j) * CH, CH)],
                                 gsem)

        def fire_scatters(slot_i, slot_r):
            for j in range(BLK):
                pltpu.async_copy(rows.at[pl.ds((slot_r * BLK + j) * CH, CH)],
                                 acc.at[didx.at[slot_i * BLK + j]],
                                 ssem, add=True)

        def drain(sem, k):
            for _ in range(k):
                pltpu.make_async_copy(h_hbm.at[pl.ds(0, CH)],
                                      rows.at[pl.ds(0, CH)], sem).wait()

        def drain_idx():
            for ref in (sidx, didx):
                pltpu.make_async_copy(srcr_hbm.at[pl.ds(0, BLK)],
                                      ref.at[pl.ds(0, BLK)], isem).wait()

        idx_copy_sync(0, 0)
        fire_gathers(0, 0)
        idx_copy_sync(1, 1)

        def loop_body(m, carry):
            si = lax.rem(m, 3)
            sr = lax.rem(m, 2)
            sin_ = lax.rem(m + 1, 3)
            srn = lax.rem(m + 1, 2)
            sif = lax.rem(m + 2, 3)

            @pl.when(m >= 1)
            def _():
                drain(ssem, BLK)
                drain_idx()

            fire_gathers(sin_, srn)
            drain(gsem, BLK)
            fire_scatters(si, sr)

            @pl.when(m < m_full - 2)
            def _():
                idx_copy_async(m + 2, sif)

            return carry

        lax.fori_loop(0, m_full - 1, loop_body, 0)

        drain(ssem, BLK)
        drain(gsem, BLK)
        fire_scatters(lax.rem(m_full - 1, 3), lax.rem(m_full - 1, 2))
        drain(ssem, BLK)

        def tail_body(t, carry):
            cb = lo + m_full * BLK + t
            pltpu.sync_copy(srcr_hbm.at[pl.ds(cb, 1)], sidx.at[pl.ds(0, 1)])
            pltpu.sync_copy(dstr_hbm.at[pl.ds(cb, 1)], didx.at[pl.ds(0, 1)])
            pltpu.async_copy(h_hbm.at[sidx.at[0]],
                             rows.at[pl.ds(0, CH)], gsem).wait()
            pltpu.async_copy(rows.at[pl.ds(0, CH)],
                             acc.at[didx.at[0]], ssem, add=True).wait()
            return carry

        lax.fori_loop(0, tail, tail_body, 0)
        plsc.subcore_barrier()

        pltpu.sync_copy(acc.at[pl.ds(s * zrows, zrows)],
                        out_hbm.at[c].at[pl.ds(s * zrows, zrows)])

    return pl.kernel(
        body,
        out_type=jax.ShapeDtypeStruct((NC, nacc, D), jnp.float32),
        mesh=mesh,
        scratch_types=[
            pltpu.VMEM((3 * BLK, CH), jnp.int32),
            pltpu.VMEM((3 * BLK, CH), jnp.int32),
            pltpu.VMEM((2 * BLK * CH, D), jnp.float32),
            pltpu.VMEM_SHARED((nacc, D), jnp.float32),
            pltpu.SemaphoreType.DMA,
            pltpu.SemaphoreType.DMA,
            pltpu.SemaphoreType.DMA,
        ],
        compiler_params=pltpu.CompilerParams(use_tc_tiling_on_sc=False),
    )


def _make_pick(p: int):
    mesh = plsc.VectorSubcoreMesh(core_axis_name="c", subcore_axis_name="s")
    fch = p // CH
    tail = p - fch * CH
    cmax = (fch + NW - 1) // NW + 1

    def body(h_hbm, idx_hbm, out_hbm, sidx, tidx, rows, gsem, isem):
        c = lax.axis_index("c")
        s = lax.axis_index("s")
        w = c * NS + s
        lo = w * fch // NW
        hi = (w + 1) * fch // NW
        cnt = hi - lo

        pltpu.sync_copy(idx_hbm.at[pl.ds(lo * CH, CH)], sidx.at[0])

        def fire(k, carry):
            @pl.when(k + 1 < cnt)
            def _():
                pltpu.async_copy(idx_hbm.at[pl.ds((lo + k + 1) * CH, CH)],
                                 sidx.at[k + 1], isem)

            pltpu.async_copy(h_hbm.at[sidx.at[k]],
                             rows.at[pl.ds(k * CH, CH)], gsem)

            @pl.when(k + 1 < cnt)
            def _():
                pltpu.make_async_copy(idx_hbm.at[pl.ds(0, CH)],
                                      sidx.at[0], isem).wait()

            return carry

        lax.fori_loop(0, cnt, fire, 0)

        if tail:
            @pl.when((c == NC - 1) & (s == NS - 1))
            def _():
                pltpu.sync_copy(idx_hbm.at[pl.ds(fch * CH, tail)], tidx)
                pltpu.async_copy(h_hbm.at[tidx],
                                 rows.at[pl.ds(cmax * CH, tail)], isem).wait()
                pltpu.sync_copy(rows.at[pl.ds(cmax * CH, tail)],
                                out_hbm.at[pl.ds(fch * CH, tail)])

        def drain(k, carry):
            pltpu.make_async_copy(h_hbm.at[pl.ds(0, CH)],
                                  rows.at[pl.ds(0, CH)], gsem).wait()
            return carry

        lax.fori_loop(0, cnt, drain, 0)

        def put(k, carry):
            pltpu.sync_copy(rows.at[pl.ds(k * CH, CH)],
                            out_hbm.at[pl.ds((lo + k) * CH, CH)])
            return carry

        lax.fori_loop(0, cnt, put, 0)

    return pl.kernel(
        body,
        out_type=jax.ShapeDtypeStruct((p, D), jnp.float32),
        mesh=mesh,
        scratch_types=[
            pltpu.VMEM((cmax, CH), jnp.int32),
            pltpu.VMEM((max(tail, 8),), jnp.int32),
            pltpu.VMEM(((cmax + 1) * CH, D), jnp.float32),
            pltpu.SemaphoreType.DMA,
            pltpu.SemaphoreType.DMA,
        ],
        compiler_params=pltpu.CompilerParams(use_tc_tiling_on_sc=False),
    )


def _enc_body(z_ref, w_ref, c_ref, o_ref):
    acc = c_ref[...]
    for k in range(8):
        acc = acc + jnp.dot(z_ref[:, k, :], w_ref[k],
                            preferred_element_type=jnp.float32)
    o_ref[...] = acc[:, None, :]


def _lin_body(p_ref, w_ref, s_ref, c_ref, o_ref, *, mean, relu_on):
    a = p_ref[0] + p_ref[1]
    if mean:
        deg = jnp.dot(a, s_ref[...], preferred_element_type=jnp.float32)
        a = a / jnp.maximum(deg, 1.0)
    y = jnp.dot(a, w_ref[...], preferred_element_type=jnp.float32) + c_ref[...]
    if relu_on:
        y = jnp.maximum(y, 0.0)
    o_ref[...] = y


def _dec_body(p_ref, w_ref, c_ref, o_ref):
    y = (jnp.dot(p_ref[...], w_ref[...],
                 preferred_element_type=jnp.float32) + c_ref[...])
    rows = o_ref.shape[0]
    for k in range(8):
        yk = y[:, 16 * k:16 * (k + 1)]
        m = jnp.max(yk, axis=1, keepdims=True)
        e = jnp.exp(yk - m)
        pk = e / jnp.sum(e, axis=1, keepdims=True)
        o_ref[:, k, :] = pk[:rows, :10]


def _pad_w(w):
    wp = jnp.zeros((D, D), jnp.float32)
    return wp.at[:w.shape[0], :w.shape[1]].set(w)


def _cvec(b, ones_lane=True):
    c = jnp.zeros((1, D), jnp.float32).at[0, :b.shape[0]].set(b)
    if ones_lane:
        c = c.at[0, 10].set(1.0)
    return c


def kernel(x, z, edge_index, z1edge_index, z2edge_index, z3edge_index,
           edge_attr, pickable, W_enc, b_enc, W_c1, b_c1, W_c2, b_c2,
           W_x1, b_x1, W_lin, b_lin):
    n, zf = z.shape
    e = z1edge_index.shape[1]
    p = pickable.shape[0]
    assert n % 8 == 0 and e % CH == 0

    nacc = _round_up(n, 256)
    nrp = nacc * D // 128
    f32 = jnp.float32

    tch = e // CH
    srcr = z1edge_index[0].reshape(tch, CH)
    dstr = z1edge_index[1].reshape(tch, CH)

    wencp = jnp.zeros((zf, D), f32).at[:, :W_enc.shape[1]].set(W_enc)
    wenc8 = jnp.zeros((8, zf, 128), f32)
    for k in range(8):
        wenc8 = wenc8.at[k, :, D * k:D * (k + 1)].set(wencp)
    eye8 = jnp.eye(8, dtype=f32)

    def w128(w):
        return jnp.kron(eye8, _pad_w(w))

    def c128(b, ones_lane=True):
        return jnp.tile(_cvec(b, ones_lane), (1, 8))

    cenc = c128(b_enc)
    w1, c1 = w128(W_c1), c128(b_c1)
    w2, c2 = w128(W_c2), c128(b_c2)
    wx, cx = w128(W_x1), c128(b_x1)
    wl = w128(W_lin)
    clv = jnp.full((1, D), -1e30, f32).at[0, :b_lin.shape[0]].set(b_lin)
    cl = jnp.tile(clv, (1, 8))
    s128 = jnp.zeros((128, 128), f32)
    for k in range(8):
        s128 = s128.at[D * k + 10, D * k:D * (k + 1)].set(1.0)

    grp = nrp // 4
    z3 = z.reshape(n // 8, 8, zf)
    enc_bl = nrp // 8

    h0p = pl.pallas_call(
        _enc_body,
        grid=(8,),
        in_specs=[pl.BlockSpec((enc_bl, 8, zf), lambda i: (i, 0, 0)),
                  pl.BlockSpec((8, zf, 128), lambda i: (0, 0, 0)),
                  pl.BlockSpec((1, 128), lambda i: (0, 0))],
        out_specs=pl.BlockSpec((enc_bl, 1, 128), lambda i: (i, 0, 0)),
        out_shape=jax.ShapeDtypeStruct((nrp, 1, 128), f32),
    )(z3, wenc8, cenc)
    h0 = h0p.reshape(nacc, D)

    def lin(mean, relu_on):
        return pl.pallas_call(
            functools.partial(_lin_body, mean=mean, relu_on=relu_on),
            grid=(4,),
            in_specs=[pl.BlockSpec((NC, grp, 128), lambda i: (0, i, 0)),
                      pl.BlockSpec((128, 128), lambda i: (0, 0)),
                      pl.BlockSpec((128, 128), lambda i: (0, 0)),
                      pl.BlockSpec((1, 128), lambda i: (0, 0))],
            out_specs=pl.BlockSpec((grp, 128), lambda i: (i, 0)),
            out_shape=jax.ShapeDtypeStruct((nrp, 128), f32),
        )

    agg = _make_agg(tch, nacc)

    def layer(h, mean, relu_on, w, cv):
        prt = agg(h, srcr, dstr)
        prtp = prt.reshape(NC, nrp, 128)
        return lin(mean, relu_on)(prtp, w, s128, cv).reshape(nacc, D)

    h1 = layer(h0, False, True, w1, c1)
    h2 = layer(h1, False, False, w2, c2)
    h3 = layer(h2, True, True, wx, cx)

    assert p % 8 == 0
    picked = _make_pick(p)(h3, pickable)

    prp = p * D // 128
    dbl = 800
    dgrid = (prp + dbl - 1) // dbl
    out3 = pl.pallas_call(
        _dec_body,
        grid=(dgrid,),
        in_specs=[pl.BlockSpec((dbl, 128), lambda i: (i, 0)),
                  pl.BlockSpec((128, 128), lambda i: (0, 0)),
                  pl.BlockSpec((1, 128), lambda i: (0, 0))],
        out_specs=pl.BlockSpec((dbl, 8, 10), lambda i: (i, 0, 0)),
        out_shape=jax.ShapeDtypeStruct((p // 8, 8, 10), f32),
    )(picked.reshape(prp, 128), wl, cl)
    return out3.reshape(p, 10)


# speedup vs baseline: 61.9705x; 1.0793x over previous
import functools

import jax
import jax.numpy as jnp
from jax import lax
from jax.experimental import pallas as pl
from jax.experimental.pallas import tpu as pltpu
from jax.experimental.pallas import tpu_sc as plsc

D = 16
CH = 128
BLK = 6
PBLK = 4
NC = 2
NS = 16
NW = NC * NS
ZCH = 512
F0_NUM, F0_DEN = 39, 100


def _round_up(a: int, b: int) -> int:
    return (a + b - 1) // b * b


def _make_agg(tch: int, nacc: int):
    mesh = plsc.VectorSubcoreMesh(core_axis_name="c", subcore_axis_name="s")
    zrows = nacc // NS
    ch0 = tch * F0_NUM // F0_DEN
    ch1 = tch - ch0
    assert min(ch0, ch1) // NS >= 3 * BLK

    def body(h_hbm, srcr_hbm, dstr_hbm, out_hbm,
             sidx, didx, rows, acc, gsem, ssem, isem):
        c = lax.axis_index("c")
        s = lax.axis_index("s")

        zvec = jnp.zeros((D,), jnp.float32)

        def zfill(i, carry):
            rows[i] = zvec
            return carry

        lax.fori_loop(0, ZCH, zfill, 0)
        zbase = s * zrows
        nfull = zrows // ZCH
        for k in range(nfull):
            pltpu.sync_copy(rows.at[pl.ds(0, ZCH)],
                            acc.at[pl.ds(zbase + k * ZCH, ZCH)])
        ztail = zrows - nfull * ZCH
        if ztail:
            pltpu.sync_copy(rows.at[pl.ds(0, ztail)],
                            acc.at[pl.ds(zbase + nfull * ZCH, ztail)])
        plsc.subcore_barrier()

        cch = jnp.where(c == 0, ch0, ch1)
        cbase = c * ch0
        lo = cbase + s * cch // NS
        hi = cbase + (s + 1) * cch // NS
        m_full = (hi - lo) // BLK
        tail = (hi - lo) - m_full * BLK

        def idx_copy_sync(m, slot):
            cb = lo + m * BLK
            pltpu.sync_copy(srcr_hbm.at[pl.ds(cb, BLK)],
                            sidx.at[pl.ds(slot * BLK, BLK)])
            pltpu.sync_copy(dstr_hbm.at[pl.ds(cb, BLK)],
                            didx.at[pl.ds(slot * BLK, BLK)])

        def idx_copy_async(m, slot):
            cb = lo + m * BLK
            pltpu.async_copy(srcr_hbm.at[pl.ds(cb, BLK)],
                             sidx.at[pl.ds(slot * BLK, BLK)], isem)
            pltpu.async_copy(dstr_hbm.at[pl.ds(cb, BLK)],
                             didx.at[pl.ds(slot * BLK, BLK)], isem)

        def fire_gathers(slot_i, slot_r):
            for j in range(BLK):
                pltpu.async_copy(h_hbm.at[sidx.at[slot_i * BLK + j]],
                                 rows.at[pl.ds((slot_r * BLK + j) * CH, CH)],
                                 gsem)

        def fire_scatters(slot_i, slot_r):
            for j in range(BLK):
                pltpu.async_copy(rows.at[pl.ds((slot_r * BLK + j) * CH, CH)],
                                 acc.at[didx.at[slot_i * BLK + j]],
                                 ssem, add=True)

        def drain(sem, k):
            for _ in range(k):
                pltpu.make_async_copy(h_hbm.at[pl.ds(0, CH)],
                                      rows.at[pl.ds(0, CH)], sem).wait()

        def drain_idx():
            for ref in (sidx, didx):
                pltpu.make_async_copy(srcr_hbm.at[pl.ds(0, BLK)],
                                      ref.at[pl.ds(0, BLK)], isem).wait()

        idx_copy_sync(0, 0)
        fire_gathers(0, 0)
        idx_copy_sync(1, 1)

        def loop_body(m, carry):
            si = lax.rem(m, 3)
            sr = lax.rem(m, 2)
            sin_ = lax.rem(m + 1, 3)
            srn = lax.rem(m + 1, 2)
            sif = lax.rem(m + 2, 3)

            @pl.when(m >= 1)
            def _():
                drain(ssem, BLK)
                drain_idx()

            fire_gathers(sin_, srn)
            drain(gsem, BLK)
            fire_scatters(si, sr)

            @pl.when(m < m_full - 2)
            def _():
                idx_copy_async(m + 2, sif)

            return carry

        lax.fori_loop(0, m_full - 1, loop_body, 0)

        drain(ssem, BLK)
        drain(gsem, BLK)
        fire_scatters(lax.rem(m_full - 1, 3), lax.rem(m_full - 1, 2))
        drain(ssem, BLK)

        def tail_body(t, carry):
            cb = lo + m_full * BLK + t
            pltpu.sync_copy(srcr_hbm.at[pl.ds(cb, 1)], sidx.at[pl.ds(0, 1)])
            pltpu.sync_copy(dstr_hbm.at[pl.ds(cb, 1)], didx.at[pl.ds(0, 1)])
            pltpu.async_copy(h_hbm.at[sidx.at[0]],
                             rows.at[pl.ds(0, CH)], gsem).wait()
            pltpu.async_copy(rows.at[pl.ds(0, CH)],
                             acc.at[didx.at[0]], ssem, add=True).wait()
            return carry

        lax.fori_loop(0, tail, tail_body, 0)
        plsc.subcore_barrier()

        pltpu.sync_copy(acc.at[pl.ds(s * zrows, zrows)],
                        out_hbm.at[c].at[pl.ds(s * zrows, zrows)])

    return pl.kernel(
        body,
        out_type=jax.ShapeDtypeStruct((NC, nacc, D), jnp.float32),
        mesh=mesh,
        scratch_types=[
            pltpu.VMEM((3 * BLK, CH), jnp.int32),
            pltpu.VMEM((3 * BLK, CH), jnp.int32),
            pltpu.VMEM((2 * BLK * CH, D), jnp.float32),
            pltpu.VMEM_SHARED((nacc, D), jnp.float32),
            pltpu.SemaphoreType.DMA,
            pltpu.SemaphoreType.DMA,
            pltpu.SemaphoreType.DMA,
        ],
        compiler_params=pltpu.CompilerParams(use_tc_tiling_on_sc=False),
    )


def _make_pick(p: int):
    mesh = plsc.VectorSubcoreMesh(core_axis_name="c", subcore_axis_name="s")
    fch = p // CH
    tail = p - fch * CH
    cmax = (fch + NW - 1) // NW + 1

    def body(h_hbm, idx_hbm, out_hbm, sidx, tidx, rows, gsem, isem):
        c = lax.axis_index("c")
        s = lax.axis_index("s")
        w = c * NS + s
        lo = w * fch // NW
        hi = (w + 1) * fch // NW
        cnt = hi - lo

        pltpu.sync_copy(idx_hbm.at[pl.ds(lo * CH, CH)], sidx.at[0])

        def fire(k, carry):
            @pl.when(k + 1 < cnt)
            def _():
                pltpu.async_copy(idx_hbm.at[pl.ds((lo + k + 1) * CH, CH)],
                                 sidx.at[k + 1], isem)

            pltpu.async_copy(h_hbm.at[sidx.at[k]],
                             rows.at[pl.ds(k * CH, CH)], gsem)

            @pl.when(k + 1 < cnt)
            def _():
                pltpu.make_async_copy(idx_hbm.at[pl.ds(0, CH)],
                                      sidx.at[0], isem).wait()

            return carry

        lax.fori_loop(0, cnt, fire, 0)

        if tail:
            @pl.when((c == NC - 1) & (s == NS - 1))
            def _():
                pltpu.sync_copy(idx_hbm.at[pl.ds(fch * CH, tail)], tidx)
                pltpu.async_copy(h_hbm.at[tidx],
                                 rows.at[pl.ds(cmax * CH, tail)], isem).wait()
                pltpu.sync_copy(rows.at[pl.ds(cmax * CH, tail)],
                                out_hbm.at[pl.ds(fch * CH, tail)])

        def drain(k, carry):
            pltpu.make_async_copy(h_hbm.at[pl.ds(0, CH)],
                                  rows.at[pl.ds(0, CH)], gsem).wait()
            return carry

        lax.fori_loop(0, cnt, drain, 0)

        def put(k, carry):
            pltpu.sync_copy(rows.at[pl.ds(k * CH, CH)],
                            out_hbm.at[pl.ds((lo + k) * CH, CH)])
            return carry

        lax.fori_loop(0, cnt, put, 0)

    return pl.kernel(
        body,
        out_type=jax.ShapeDtypeStruct((p, D), jnp.float32),
        mesh=mesh,
        scratch_types=[
            pltpu.VMEM((cmax, CH), jnp.int32),
            pltpu.VMEM((max(tail, 8),), jnp.int32),
            pltpu.VMEM(((cmax + 1) * CH, D), jnp.float32),
            pltpu.SemaphoreType.DMA,
            pltpu.SemaphoreType.DMA,
        ],
        compiler_params=pltpu.CompilerParams(use_tc_tiling_on_sc=False),
    )


def _enc_body(z_ref, w_ref, c_ref, o_ref):
    acc = c_ref[...]
    for k in range(8):
        acc = acc + jnp.dot(z_ref[:, k, :], w_ref[k],
                            preferred_element_type=jnp.float32)
    o_ref[...] = acc[:, None, :]


def _lin_body(p_ref, w_ref, s_ref, c_ref, o_ref, *, mean, relu_on):
    a = p_ref[0] + p_ref[1]
    if mean:
        deg = jnp.dot(a, s_ref[...], preferred_element_type=jnp.float32)
        a = a / jnp.maximum(deg, 1.0)
    y = jnp.dot(a, w_ref[...], preferred_element_type=jnp.float32) + c_ref[...]
    if relu_on:
        y = jnp.maximum(y, 0.0)
    o_ref[...] = y


def _dec_body(p_ref, w_ref, c_ref, o_ref):
    y = (jnp.dot(p_ref[...], w_ref[...],
                 preferred_element_type=jnp.float32) + c_ref[...])
    rows = o_ref.shape[0]
    for k in range(8):
        yk = y[:, 16 * k:16 * (k + 1)]
        m = jnp.max(yk, axis=1, keepdims=True)
        e = jnp.exp(yk - m)
        pk = e / jnp.sum(e, axis=1, keepdims=True)
        o_ref[:, k, :] = pk[:rows, :10]


def _pad_w(w):
    wp = jnp.zeros((D, D), jnp.float32)
    return wp.at[:w.shape[0], :w.shape[1]].set(w)


def _cvec(b, ones_lane=True):
    c = jnp.zeros((1, D), jnp.float32).at[0, :b.shape[0]].set(b)
    if ones_lane:
        c = c.at[0, 10].set(1.0)
    return c


def kernel(x, z, edge_index, z1edge_index, z2edge_index, z3edge_index,
           edge_attr, pickable, W_enc, b_enc, W_c1, b_c1, W_c2, b_c2,
           W_x1, b_x1, W_lin, b_lin):
    n, zf = z.shape
    e = z1edge_index.shape[1]
    p = pickable.shape[0]
    assert n % 8 == 0 and e % CH == 0

    nacc = _round_up(n, 256)
    nrp = nacc * D // 128
    f32 = jnp.float32

    tch = e // CH
    srcr = z1edge_index[0].reshape(tch, CH)
    dstr = z1edge_index[1].reshape(tch, CH)

    wencp = jnp.zeros((zf, D), f32).at[:, :W_enc.shape[1]].set(W_enc)
    wenc8 = jnp.zeros((8, zf, 128), f32)
    for k in range(8):
        wenc8 = wenc8.at[k, :, D * k:D * (k + 1)].set(wencp)
    eye8 = jnp.eye(8, dtype=f32)

    def w128(w):
        return jnp.kron(eye8, _pad_w(w))

    def c128(b, ones_lane=True):
        return jnp.tile(_cvec(b, ones_lane), (1, 8))

    cenc = c128(b_enc)
    w1, c1 = w128(W_c1), c128(b_c1)
    w2, c2 = w128(W_c2), c128(b_c2)
    wx, cx = w128(W_x1), c128(b_x1)
    wl = w128(W_lin)
    clv = jnp.full((1, D), -1e30, f32).at[0, :b_lin.shape[0]].set(b_lin)
    cl = jnp.tile(clv, (1, 8))
    s128 = jnp.zeros((128, 128), f32)
    for k in range(8):
        s128 = s128.at[D * k + 10, D * k:D * (k + 1)].set(1.0)

    grp = nrp // 4
    z3 = z.reshape(n // 8, 8, zf)
    enc_bl = nrp // 8

    h0p = pl.pallas_call(
        _enc_body,
        grid=(8,),
        in_specs=[pl.BlockSpec((enc_bl, 8, zf), lambda i: (i, 0, 0)),
                  pl.BlockSpec((8, zf, 128), lambda i: (0, 0, 0)),
                  pl.BlockSpec((1, 128), lambda i: (0, 0))],
        out_specs=pl.BlockSpec((enc_bl, 1, 128), lambda i: (i, 0, 0)),
        out_shape=jax.ShapeDtypeStruct((nrp, 1, 128), f32),
    )(z3, wenc8, cenc)
    h0 = h0p.reshape(nacc, D)

    def lin(mean, relu_on):
        return pl.pallas_call(
            functools.partial(_lin_body, mean=mean, relu_on=relu_on),
            grid=(4,),
            in_specs=[pl.BlockSpec((NC, grp, 128), lambda i: (0, i, 0)),
                      pl.BlockSpec((128, 128), lambda i: (0, 0)),
                      pl.BlockSpec((128, 128), lambda i: (0, 0)),
                      pl.BlockSpec((1, 128), lambda i: (0, 0))],
            out_specs=pl.BlockSpec((grp, 128), lambda i: (i, 0)),
            out_shape=jax.ShapeDtypeStruct((nrp, 128), f32),
        )

    agg = _make_agg(tch, nacc)

    def layer(h, mean, relu_on, w, cv):
        prt = agg(h, srcr, dstr)
        prtp = prt.reshape(NC, nrp, 128)
        return lin(mean, relu_on)(prtp, w, s128, cv).reshape(nacc, D)

    h1 = layer(h0, False, True, w1, c1)
    h2 = layer(h1, False, False, w2, c2)
    h3 = layer(h2, True, True, wx, cx)

    assert p % 8 == 0
    picked = _make_pick(p)(h3, pickable)

    prp = p * D // 128
    dbl = 800
    dgrid = (prp + dbl - 1) // dbl
    out3 = pl.pallas_call(
        _dec_body,
        grid=(dgrid,),
        in_specs=[pl.BlockSpec((dbl, 128), lambda i: (i, 0)),
                  pl.BlockSpec((128, 128), lambda i: (0, 0)),
                  pl.BlockSpec((1, 128), lambda i: (0, 0))],
        out_specs=pl.BlockSpec((dbl, 8, 10), lambda i: (i, 0, 0)),
        out_shape=jax.ShapeDtypeStruct((p // 8, 8, 10), f32),
    )(picked.reshape(prp, 128), wl, cl)
    return out3.reshape(p, 10)
